# baseline (speedup 1.0000x reference)
import jax
import jax.numpy as jnp
from jax import lax
from jax.experimental import pallas as pl
from jax.experimental.pallas import tpu as pltpu
from jax.experimental.pallas import tpu_sc as plsc

B = 8192
W = 4096
C = 4096
L = 16

NC = 2
NS = 16
NW = NC * NS

G = 8
ROWS_PER_W = B // NW
STEPS = ROWS_PER_W // G
CCHUNKS = C // L


def _body(inp_hbm, wv_hbm, idx_hbm, out_hbm, idx_v, wv_v, in_v, out_v, sem):
    cid = lax.axis_index("c")
    sid = lax.axis_index("s")
    wid = sid * NC + cid
    base = wid * ROWS_PER_W

    pltpu.sync_copy(idx_hbm, idx_v)
    pltpu.sync_copy(wv_hbm, wv_v)

    @pl.loop(0, STEPS)
    def _step(t):
        row0 = base + t * G
        for g in range(G):
            pltpu.sync_copy(inp_hbm.at[row0 + g], in_v.at[pl.ds(g * W, W)])
        copies = [
            pltpu.async_copy(
                in_v.at[pl.ds(g * W, W)], out_hbm.at[row0 + g, pl.ds(0, W)], sem
            )
            for g in range(G)
        ]

        @pl.loop(0, CCHUNKS)
        def _chunk(j):
            sl = pl.ds(j * L, L)
            ids = idx_v[sl]
            w = wv_v[sl]
            for g in range(G):
                vals = plsc.load_gather(in_v, [ids + (g * W)])
                out_v[pl.ds(g * C + j * L, L)] = vals * w

        for cp in copies:
            cp.wait()
        for g in range(G):
            pltpu.sync_copy(
                out_v.at[pl.ds(g * C, C)], out_hbm.at[row0 + g, pl.ds(W, C)]
            )


@jax.jit
def kernel(input, weight_values, parent_idx):
    mesh = plsc.VectorSubcoreMesh(core_axis_name="c", subcore_axis_name="s")
    run = pl.kernel(
        _body,
        out_type=jax.ShapeDtypeStruct((B, W + C), jnp.float32),
        mesh=mesh,
        scratch_types=[
            pltpu.VMEM((C,), jnp.int32),
            pltpu.VMEM((C,), jnp.float32),
            pltpu.VMEM((G * W,), jnp.float32),
            pltpu.VMEM((G * C,), jnp.float32),
            pltpu.SemaphoreType.DMA,
        ],
        compiler_params=pltpu.CompilerParams(needs_layout_passes=False),
    )
    return run(input, weight_values, parent_idx.astype(jnp.int32))
